# speedup vs baseline: 2.5919x; 2.5919x over previous
import functools

import jax
import jax.numpy as jnp
from jax import lax
from jax.experimental import pallas as pl

B = 512
F = 128
NBITS = 16
ENC = F * NBITS
H = 4096
K = 32
C = 1000
THRESH = 8.0


def _encode_body(x_ref, out_ref):
    x = x_ref[...]
    for t in range(NBITS):
        th = (t + 0.5) / NBITS
        out_ref[:, t * F:(t + 1) * F] = (x >= th).astype(jnp.float32)


def _layer_body(prev_ref, idx_ref, sgn_ref, out_ref, *, d, remap):
    hb = idx_ref.shape[0]
    eidx = lax.broadcasted_iota(jnp.int32, (hb, d), 1)
    acc = jnp.zeros((hb, d), jnp.float32)
    for k in range(K):
        idv = idx_ref[:, k:k + 1]
        if remap:
            idv = (idv % NBITS) * F + idv // NBITS
        sgv = sgn_ref[:, k:k + 1]
        acc = acc + jnp.where(eidx == idv, sgv, 0.0)
    s = lax.dot_general(prev_ref[...], acc, (((1,), (1,)), ((), ())),
                        preferred_element_type=jnp.float32)
    out_ref[...] = (s >= THRESH).astype(jnp.float32)


def _layer(prev, idx, sgn, *, d, remap, hb):
    grid = H // hb
    return pl.pallas_call(
        functools.partial(_layer_body, d=d, remap=remap),
        grid=(grid,),
        in_specs=[
            pl.BlockSpec((B, d), lambda j: (0, 0)),
            pl.BlockSpec((hb, K), lambda j: (j, 0)),
            pl.BlockSpec((hb, K), lambda j: (j, 0)),
        ],
        out_specs=pl.BlockSpec((B, hb), lambda j: (0, j)),
        out_shape=jax.ShapeDtypeStruct((B, H), jnp.float32),
    )(prev, idx, sgn)


def _out_body(a0_ref, a1_ref, o0_ref, o1_ref, out_ref):
    j = pl.program_id(0)
    part = (jnp.dot(a0_ref[...], o0_ref[...], preferred_element_type=jnp.float32)
            + jnp.dot(a1_ref[...], o1_ref[...], preferred_element_type=jnp.float32))

    @pl.when(j == 0)
    def _init():
        out_ref[...] = part

    @pl.when(j > 0)
    def _acc():
        out_ref[...] += part


def kernel(x, idx0, sgn0, idx1, sgn1, outW):
    idx0 = idx0.astype(jnp.int32)
    idx1 = idx1.astype(jnp.int32)
    enc = pl.pallas_call(
        _encode_body,
        out_shape=jax.ShapeDtypeStruct((B, ENC), jnp.float32),
    )(x)
    a0 = _layer(enc, idx0, sgn0, d=ENC, remap=True, hb=512)
    a1 = _layer(a0, idx1, sgn1, d=H, remap=False, hb=512)
    hb = 512
    out = pl.pallas_call(
        _out_body,
        grid=(H // hb,),
        in_specs=[
            pl.BlockSpec((B, hb), lambda j: (0, j)),
            pl.BlockSpec((B, hb), lambda j: (0, j)),
            pl.BlockSpec((hb, C), lambda j: (j, 0)),
            pl.BlockSpec((hb, C), lambda j: (j, 0)),
        ],
        out_specs=pl.BlockSpec((B, C), lambda j: (0, 0)),
        out_shape=jax.ShapeDtypeStruct((B, C), jnp.float32),
    )(a0, a1, outW[0], outW[1])
    return out


# speedup vs baseline: 4.3101x; 1.6629x over previous
import functools

import jax
import jax.numpy as jnp
from jax import lax
from jax.experimental import pallas as pl
from jax.experimental.pallas import tpu as pltpu
from jax.experimental.pallas import tpu_sc as plsc

B = 512
F = 128
NBITS = 16
ENC = F * NBITS
H = 4096
K = 32
C = 1000
THRESH = 8.0

NC = 2
NS = 16
NW = NC * NS
HPW = H // NW
R = 8
G = HPW // R

_mesh = plsc.VectorSubcoreMesh(
    core_axis_name="c", subcore_axis_name="s", num_cores=NC, num_subcores=NS)


def _build_body(idx0_hbm, sgn0_hbm, idx1_hbm, sgn1_hbm,
                w0_hbm, w1_hbm, idx_v, sgn_v, wbuf):
    wid = lax.axis_index("s") * NC + lax.axis_index("c")
    base = wid * HPW
    zero16 = jnp.zeros((16,), jnp.float32)

    def zstep(i, c):
        wbuf[pl.ds(i * 16, 16)] = zero16
        return c
    lax.fori_loop(0, (R * H) // 16, zstep, 0)

    def build_layer(idx_hbm, sgn_hbm, w_hbm, d, remap):
        pltpu.sync_copy(idx_hbm.at[pl.ds(base * K, HPW * K)], idx_v)
        pltpu.sync_copy(sgn_hbm.at[pl.ds(base * K, HPW * K)], sgn_v)

        def group(g, c):
            touched = []
            for r in range(R):
                o = (g * R + r) * K
                for half in range(2):
                    iv = idx_v[pl.ds(o + 16 * half, 16)]
                    sv = sgn_v[pl.ds(o + 16 * half, 16)]
                    if remap:
                        iv = ((iv & (NBITS - 1)) << 7) | (iv >> 4)
                    fi = iv + r * d
                    plsc.addupdate_scatter(wbuf, [fi], sv)
                    touched.append(fi)
            pltpu.sync_copy(wbuf.at[pl.ds(0, R * d)],
                            w_hbm.at[pl.ds((base + g * R) * d, R * d)])
            for fi in touched:
                plsc.store_scatter(wbuf, [fi], zero16)
            return c
        lax.fori_loop(0, G, group, 0)

    build_layer(idx0_hbm, sgn0_hbm, w0_hbm, ENC, True)
    build_layer(idx1_hbm, sgn1_hbm, w1_hbm, H, False)


_build = pl.kernel(
    _build_body,
    out_type=(jax.ShapeDtypeStruct((H * ENC,), jnp.float32),
              jax.ShapeDtypeStruct((H * H,), jnp.float32)),
    mesh=_mesh,
    compiler_params=pltpu.CompilerParams(needs_layout_passes=False),
    scratch_types=[
        pltpu.VMEM((HPW * K,), jnp.int32),
        pltpu.VMEM((HPW * K,), jnp.float32),
        pltpu.VMEM((R * H,), jnp.float32),
    ],
)


def _encode_body(x_ref, out_ref):
    x = x_ref[...]
    for t in range(NBITS):
        th = (t + 0.5) / NBITS
        out_ref[:, t * F:(t + 1) * F] = (x >= th).astype(jnp.float32)


def _mm_layer_body(prev_ref, w_ref, out_ref):
    s = lax.dot_general(prev_ref[...], w_ref[...], (((1,), (1,)), ((), ())),
                        preferred_element_type=jnp.float32)
    out_ref[...] = (s >= THRESH).astype(jnp.float32)


def _mm_layer(prev, wt, *, d, hb):
    return pl.pallas_call(
        _mm_layer_body,
        grid=(H // hb,),
        in_specs=[
            pl.BlockSpec((B, d), lambda j: (0, 0)),
            pl.BlockSpec((hb, d), lambda j: (j, 0)),
        ],
        out_specs=pl.BlockSpec((B, hb), lambda j: (0, j)),
        out_shape=jax.ShapeDtypeStruct((B, H), jnp.float32),
    )(prev, wt)


def _out_body(a0_ref, a1_ref, o0_ref, o1_ref, out_ref):
    j = pl.program_id(0)
    part = (jnp.dot(a0_ref[...], o0_ref[...], preferred_element_type=jnp.float32)
            + jnp.dot(a1_ref[...], o1_ref[...], preferred_element_type=jnp.float32))

    @pl.when(j == 0)
    def _init():
        out_ref[...] = part

    @pl.when(j > 0)
    def _acc():
        out_ref[...] += part


def kernel(x, idx0, sgn0, idx1, sgn1, outW):
    idx0f = idx0.astype(jnp.int32).reshape(-1)
    idx1f = idx1.astype(jnp.int32).reshape(-1)
    sgn0f = sgn0.reshape(-1)
    sgn1f = sgn1.reshape(-1)
    w0f, w1f = _build(idx0f, sgn0f, idx1f, sgn1f)
    enc = pl.pallas_call(
        _encode_body,
        out_shape=jax.ShapeDtypeStruct((B, ENC), jnp.float32),
    )(x)
    a0 = _mm_layer(enc, w0f.reshape(H, ENC), d=ENC, hb=512)
    a1 = _mm_layer(a0, w1f.reshape(H, H), d=H, hb=512)
    hb = 512
    out = pl.pallas_call(
        _out_body,
        grid=(H // hb,),
        in_specs=[
            pl.BlockSpec((B, hb), lambda j: (0, j)),
            pl.BlockSpec((B, hb), lambda j: (0, j)),
            pl.BlockSpec((hb, C), lambda j: (j, 0)),
            pl.BlockSpec((hb, C), lambda j: (j, 0)),
        ],
        out_specs=pl.BlockSpec((B, C), lambda j: (0, 0)),
        out_shape=jax.ShapeDtypeStruct((B, C), jnp.float32),
    )(a0, a1, outW[0], outW[1])
    return out


# speedup vs baseline: 4.4317x; 1.0282x over previous
import functools

import jax
import jax.numpy as jnp
from jax import lax
from jax.experimental import pallas as pl
from jax.experimental.pallas import tpu as pltpu
from jax.experimental.pallas import tpu_sc as plsc

B = 512
F = 128
NBITS = 16
ENC = F * NBITS
H = 4096
K = 32
C = 1000
THRESH = 8.0

NC = 2
NS = 16
NW = NC * NS
HPW = H // NW
R = 8
G = HPW // R

_mesh = plsc.VectorSubcoreMesh(
    core_axis_name="c", subcore_axis_name="s", num_cores=NC, num_subcores=NS)


def _build_body(idx_hbm, sgn_hbm, w_hbm, idx_v, sgn_v, wbuf, *, d, remap):
    wid = lax.axis_index("s") * NC + lax.axis_index("c")
    base = wid * HPW
    zero16 = jnp.zeros((16,), jnp.float32)

    def zstep(i, c):
        wbuf[pl.ds(i * 16, 16)] = zero16
        return c
    lax.fori_loop(0, (R * d) // 16, zstep, 0)

    pltpu.sync_copy(idx_hbm.at[pl.ds(base * K, HPW * K)], idx_v)
    pltpu.sync_copy(sgn_hbm.at[pl.ds(base * K, HPW * K)], sgn_v)

    def group(g, c):
        touched = []
        for r in range(R):
            o = (g * R + r) * K
            for half in range(2):
                iv = idx_v[pl.ds(o + 16 * half, 16)]
                sv = sgn_v[pl.ds(o + 16 * half, 16)]
                if remap:
                    iv = ((iv & (NBITS - 1)) << 7) | (iv >> 4)
                fi = iv + r * d
                plsc.addupdate_scatter(wbuf, [fi], sv)
                touched.append(fi)
        pltpu.sync_copy(wbuf.at[pl.ds(0, R * d)],
                        w_hbm.at[pl.ds((base + g * R) * d, R * d)])
        for fi in touched:
            plsc.store_scatter(wbuf, [fi], zero16)
        return c
    lax.fori_loop(0, G, group, 0)


def _make_build(d, remap):
    return pl.kernel(
        functools.partial(_build_body, d=d, remap=remap),
        out_type=jax.ShapeDtypeStruct((H * d,), jnp.float32),
        mesh=_mesh,
        compiler_params=pltpu.CompilerParams(needs_layout_passes=False),
        scratch_types=[
            pltpu.VMEM((HPW * K,), jnp.int32),
            pltpu.VMEM((HPW * K,), jnp.float32),
            pltpu.VMEM((R * d,), jnp.float32),
        ],
    )


_build0 = _make_build(ENC, True)
_build1 = _make_build(H, False)


def _encode_body(x_ref, out_ref):
    x = x_ref[...]
    for t in range(NBITS):
        th = (t + 0.5) / NBITS
        out_ref[:, t * F:(t + 1) * F] = (x >= th).astype(jnp.bfloat16)


def _mm0_body(enc_ref, w_ref, out_ref):
    s = lax.dot_general(enc_ref[...], w_ref[...].astype(jnp.bfloat16),
                        (((1,), (1,)), ((), ())),
                        preferred_element_type=jnp.float32)
    out_ref[...] = (s >= THRESH).astype(jnp.float32)


def _mm1_out_body(a0f_ref, a0b_ref, w_ref, o0_ref, o1_ref, out_ref):
    j = pl.program_id(0)
    s = lax.dot_general(a0f_ref[...].astype(jnp.bfloat16),
                        w_ref[...].astype(jnp.bfloat16),
                        (((1,), (1,)), ((), ())),
                        preferred_element_type=jnp.float32)
    a1b = (s >= THRESH).astype(jnp.float32)
    part = (jnp.dot(a0b_ref[...], o0_ref[...], preferred_element_type=jnp.float32)
            + jnp.dot(a1b, o1_ref[...], preferred_element_type=jnp.float32))

    @pl.when(j == 0)
    def _init():
        out_ref[...] = part

    @pl.when(j > 0)
    def _acc():
        out_ref[...] += part


def kernel(x, idx0, sgn0, idx1, sgn1, outW):
    idx0f = idx0.astype(jnp.int32).reshape(-1)
    idx1f = idx1.astype(jnp.int32).reshape(-1)
    sgn0f = sgn0.reshape(-1)
    sgn1f = sgn1.reshape(-1)
    w1f = _build1(idx1f, sgn1f)
    w0f = _build0(idx0f, sgn0f)
    enc = pl.pallas_call(
        _encode_body,
        out_shape=jax.ShapeDtypeStruct((B, ENC), jnp.bfloat16),
    )(x)
    hb = 512
    a0 = pl.pallas_call(
        _mm0_body,
        grid=(H // hb,),
        in_specs=[
            pl.BlockSpec((B, ENC), lambda j: (0, 0)),
            pl.BlockSpec((hb, ENC), lambda j: (j, 0)),
        ],
        out_specs=pl.BlockSpec((B, hb), lambda j: (0, j)),
        out_shape=jax.ShapeDtypeStruct((B, H), jnp.float32),
    )(enc, w0f.reshape(H, ENC))
    out = pl.pallas_call(
        _mm1_out_body,
        grid=(H // hb,),
        in_specs=[
            pl.BlockSpec((B, H), lambda j: (0, 0)),
            pl.BlockSpec((B, hb), lambda j: (0, j)),
            pl.BlockSpec((hb, H), lambda j: (j, 0)),
            pl.BlockSpec((hb, C), lambda j: (j, 0)),
            pl.BlockSpec((hb, C), lambda j: (j, 0)),
        ],
        out_specs=pl.BlockSpec((B, C), lambda j: (0, 0)),
        out_shape=jax.ShapeDtypeStruct((B, C), jnp.float32),
    )(a0, a0, w1f.reshape(H, H), outW[0], outW[1])
    return out


# speedup vs baseline: 7.0115x; 1.5821x over previous
import functools

import jax
import jax.numpy as jnp
from jax import lax
from jax.experimental import pallas as pl
from jax.experimental.pallas import tpu as pltpu
from jax.experimental.pallas import tpu_sc as plsc

B = 512
F = 128
NBITS = 16
ENC = F * NBITS
H = 4096
K = 32
C = 1000
THRESH = 8.0

NC = 2
NS = 16
NW = NC * NS
HPW = H // NW
R = 8
G = HPW // R

_mesh = plsc.VectorSubcoreMesh(
    core_axis_name="c", subcore_axis_name="s", num_cores=NC, num_subcores=NS)


def _build_body(idx_hbm, sgn_hbm, w_hbm, idx_v, sgn_v, wbuf, *, d, remap):
    wid = lax.axis_index("s") * NC + lax.axis_index("c")
    base = wid * HPW
    shift = d.bit_length() - 1
    zero16 = jnp.zeros((16,), jnp.float32)

    def zrow(rr, c):
        def zcol(i, c2):
            wbuf[rr, pl.ds(i * 16, 16)] = zero16
            return c2
        return lax.fori_loop(0, d // 16, zcol, c)
    lax.fori_loop(0, R, zrow, 0)

    pltpu.sync_copy(idx_hbm.at[pl.ds(base * K, HPW * K)], idx_v)
    pltpu.sync_copy(sgn_hbm.at[pl.ds(base * K, HPW * K)], sgn_v)

    def group(g, c):
        touched = []
        for r in range(R):
            o = (g * R + r) * K
            for half in range(2):
                iv = idx_v[pl.ds(o + 16 * half, 16)]
                sv = sgn_v[pl.ds(o + 16 * half, 16)]
                if remap:
                    iv = ((iv & (NBITS - 1)) << 7) | (iv >> 4)
                row = jnp.full((16,), r, jnp.int32)
                col = iv
                plsc.addupdate_scatter(wbuf, [row, col], sv)
                touched.append((row, col))
        pltpu.sync_copy(wbuf, w_hbm.at[wid * G + g])
        for row, col in touched:
            plsc.store_scatter(wbuf, [row, col], zero16)
        return c
    lax.fori_loop(0, G, group, 0)


def _make_build(d, remap):
    return pl.kernel(
        functools.partial(_build_body, d=d, remap=remap),
        out_type=jax.ShapeDtypeStruct((H // 8, 8, d), jnp.float32),
        mesh=_mesh,
        compiler_params=pltpu.CompilerParams(needs_layout_passes=False),
        scratch_types=[
            pltpu.VMEM((HPW * K,), jnp.int32),
            pltpu.VMEM((HPW * K,), jnp.float32),
            pltpu.VMEM((R, d), jnp.float32),
        ],
    )


_build0 = _make_build(ENC, True)
_build1 = _make_build(H, False)


def _encode_body(x_ref, out_ref):
    x = x_ref[...]
    for t in range(NBITS):
        th = (t + 0.5) / NBITS
        out_ref[:, t * F:(t + 1) * F] = (x >= th).astype(jnp.bfloat16)


def _mm0_body(enc_ref, w_ref, out_ref):
    hb = w_ref.shape[0] * 8
    w = w_ref[...].reshape(hb, ENC)
    s = lax.dot_general(enc_ref[...], w.astype(jnp.bfloat16),
                        (((1,), (1,)), ((), ())),
                        preferred_element_type=jnp.float32)
    out_ref[...] = (s >= THRESH).astype(jnp.float32)


def _mm1_out_body(a0f_ref, a0b_ref, w_ref, o0_ref, o1_ref, out_ref):
    j = pl.program_id(0)
    hb = w_ref.shape[0] * 8
    w = w_ref[...].reshape(hb, H)
    s = lax.dot_general(a0f_ref[...].astype(jnp.bfloat16),
                        w.astype(jnp.bfloat16),
                        (((1,), (1,)), ((), ())),
                        preferred_element_type=jnp.float32)
    a1b = (s >= THRESH).astype(jnp.float32)
    part = (jnp.dot(a0b_ref[...], o0_ref[...], preferred_element_type=jnp.float32)
            + jnp.dot(a1b, o1_ref[...], preferred_element_type=jnp.float32))

    @pl.when(j == 0)
    def _init():
        out_ref[...] = part

    @pl.when(j > 0)
    def _acc():
        out_ref[...] += part


def kernel(x, idx0, sgn0, idx1, sgn1, outW):
    idx0f = idx0.astype(jnp.int32).reshape(-1)
    idx1f = idx1.astype(jnp.int32).reshape(-1)
    sgn0f = sgn0.reshape(-1)
    sgn1f = sgn1.reshape(-1)
    w0t = _build0(idx0f, sgn0f)
    w1t = _build1(idx1f, sgn1f)
    enc = pl.pallas_call(
        _encode_body,
        out_shape=jax.ShapeDtypeStruct((B, ENC), jnp.bfloat16),
    )(x)
    hb = 512
    a0 = pl.pallas_call(
        _mm0_body,
        grid=(H // hb,),
        in_specs=[
            pl.BlockSpec((B, ENC), lambda j: (0, 0)),
            pl.BlockSpec((hb // 8, 8, ENC), lambda j: (j, 0, 0)),
        ],
        out_specs=pl.BlockSpec((B, hb), lambda j: (0, j)),
        out_shape=jax.ShapeDtypeStruct((B, H), jnp.float32),
    )(enc, w0t)
    out = pl.pallas_call(
        _mm1_out_body,
        grid=(H // hb,),
        in_specs=[
            pl.BlockSpec((B, H), lambda j: (0, 0)),
            pl.BlockSpec((B, hb), lambda j: (0, j)),
            pl.BlockSpec((hb // 8, 8, H), lambda j: (j, 0, 0)),
            pl.BlockSpec((hb, C), lambda j: (j, 0)),
            pl.BlockSpec((hb, C), lambda j: (j, 0)),
        ],
        out_specs=pl.BlockSpec((B, C), lambda j: (0, 0)),
        out_shape=jax.ShapeDtypeStruct((B, C), jnp.float32),
    )(a0, a0, w1t, outW[0], outW[1])
    return out


# speedup vs baseline: 8.2591x; 1.1779x over previous
import functools

import jax
import jax.numpy as jnp
from jax import lax
from jax.experimental import pallas as pl
from jax.experimental.pallas import tpu as pltpu
from jax.experimental.pallas import tpu_sc as plsc

B = 512
F = 128
NBITS = 16
ENC = F * NBITS
H = 4096
K = 32
C = 1000
THRESH = 8.0

NC = 2
NS = 16
NW = NC * NS
HPW = H // NW
R = 8
G = HPW // R

_mesh = plsc.VectorSubcoreMesh(
    core_axis_name="c", subcore_axis_name="s", num_cores=NC, num_subcores=NS)


def _build_body(idx_hbm, sgn_hbm, w_hbm, idx_v, sgn_v, wbuf, *, d, remap):
    wid = lax.axis_index("s") * NC + lax.axis_index("c")
    base = wid * HPW
    shift = d.bit_length() - 1
    zero16 = jnp.zeros((16,), jnp.float32)

    def zrow(rr, c):
        def zcol(i, c2):
            wbuf[rr, pl.ds(i * 16, 16)] = zero16
            return c2
        return lax.fori_loop(0, d // 16, zcol, c)
    lax.fori_loop(0, R, zrow, 0)

    pltpu.sync_copy(idx_hbm.at[pl.ds(base, HPW)], idx_v)
    pltpu.sync_copy(sgn_hbm.at[pl.ds(base, HPW)], sgn_v)

    def group(g, c):
        touched = []
        for r in range(R):
            for half in range(2):
                iv = idx_v[g * R + r, pl.ds(16 * half, 16)]
                sv = sgn_v[g * R + r, pl.ds(16 * half, 16)]
                if remap:
                    iv = ((iv & (NBITS - 1)) << 7) | (iv >> 4)
                row = jnp.full((16,), r, jnp.int32)
                col = iv
                plsc.addupdate_scatter(wbuf, [row, col], sv)
                touched.append((row, col))
        pltpu.sync_copy(wbuf, w_hbm.at[wid * G + g])
        for row, col in touched:
            plsc.store_scatter(wbuf, [row, col], zero16)
        return c
    lax.fori_loop(0, G, group, 0)


def _make_build(d, remap):
    return pl.kernel(
        functools.partial(_build_body, d=d, remap=remap),
        out_type=jax.ShapeDtypeStruct((H // 8, 8, d), jnp.float32),
        mesh=_mesh,
        compiler_params=pltpu.CompilerParams(needs_layout_passes=False),
        scratch_types=[
            pltpu.VMEM((HPW, K), jnp.int32),
            pltpu.VMEM((HPW, K), jnp.float32),
            pltpu.VMEM((R, d), jnp.float32),
        ],
    )


_build0 = _make_build(ENC, True)
_build1 = _make_build(H, False)


def _encode_body(x_ref, out_ref):
    x = x_ref[...]
    for t in range(NBITS):
        th = (t + 0.5) / NBITS
        out_ref[:, t * F:(t + 1) * F] = (x >= th).astype(jnp.bfloat16)


def _mm0_body(enc_ref, w_ref, out_ref):
    hb = w_ref.shape[0] * 8
    w = w_ref[...].reshape(hb, ENC)
    s = lax.dot_general(enc_ref[...], w.astype(jnp.bfloat16),
                        (((1,), (1,)), ((), ())),
                        preferred_element_type=jnp.float32)
    out_ref[...] = (s >= THRESH).astype(jnp.float32)


def _mm1_out_body(a0f_ref, a0b_ref, w_ref, o0_ref, o1_ref, out_ref):
    j = pl.program_id(0)
    hb = w_ref.shape[0] * 8
    w = w_ref[...].reshape(hb, H)
    s = lax.dot_general(a0f_ref[...].astype(jnp.bfloat16),
                        w.astype(jnp.bfloat16),
                        (((1,), (1,)), ((), ())),
                        preferred_element_type=jnp.float32)
    a1b = (s >= THRESH).astype(jnp.float32)
    o0 = o0_ref[...].reshape(hb, C)
    o1 = o1_ref[...].reshape(hb, C)
    part = (jnp.dot(a0b_ref[...], o0, preferred_element_type=jnp.float32)
            + jnp.dot(a1b, o1, preferred_element_type=jnp.float32))

    @pl.when(j == 0)
    def _init():
        out_ref[...] = part

    @pl.when(j > 0)
    def _acc():
        out_ref[...] += part


def kernel(x, idx0, sgn0, idx1, sgn1, outW):
    w0t = _build0(idx0.astype(jnp.int32), sgn0)
    w1t = _build1(idx1.astype(jnp.int32), sgn1)
    enc = pl.pallas_call(
        _encode_body,
        out_shape=jax.ShapeDtypeStruct((B, ENC), jnp.bfloat16),
    )(x)
    hb = 512
    a0 = pl.pallas_call(
        _mm0_body,
        grid=(H // hb,),
        in_specs=[
            pl.BlockSpec((B, ENC), lambda j: (0, 0)),
            pl.BlockSpec((hb // 8, 8, ENC), lambda j: (j, 0, 0)),
        ],
        out_specs=pl.BlockSpec((B, hb), lambda j: (0, j)),
        out_shape=jax.ShapeDtypeStruct((B, H), jnp.float32),
    )(enc, w0t)
    out = pl.pallas_call(
        _mm1_out_body,
        grid=(H // hb,),
        in_specs=[
            pl.BlockSpec((B, H), lambda j: (0, 0)),
            pl.BlockSpec((B, hb), lambda j: (0, j)),
            pl.BlockSpec((hb // 8, 8, H), lambda j: (j, 0, 0)),
            pl.BlockSpec((1, hb, C), lambda j: (0, j, 0)),
            pl.BlockSpec((1, hb, C), lambda j: (1, j, 0)),
        ],
        out_specs=pl.BlockSpec((B, C), lambda j: (0, 0)),
        out_shape=jax.ShapeDtypeStruct((B, C), jnp.float32),
    )(a0, a0, w1t, outW, outW)
    return out


# speedup vs baseline: 9.6159x; 1.1643x over previous
import functools

import jax
import jax.numpy as jnp
from jax import lax
from jax.experimental import pallas as pl
from jax.experimental.pallas import tpu as pltpu
from jax.experimental.pallas import tpu_sc as plsc

B = 512
F = 128
NBITS = 16
ENC = F * NBITS
H = 4096
K = 32
C = 1000
THRESH = 8.0

NC = 2
NS = 16
NW = NC * NS
HPW = H // NW
R = 8
G = HPW // R

_mesh = plsc.VectorSubcoreMesh(
    core_axis_name="c", subcore_axis_name="s", num_cores=NC, num_subcores=NS)


def _build_body(idx_hbm, sgn_hbm, w_hbm, idx_v, sgn_v, wbuf, *, d, remap):
    wid = lax.axis_index("s") * NC + lax.axis_index("c")
    base = wid * HPW
    shift = d.bit_length() - 1
    zero16 = jnp.zeros((16,), jnp.float32)

    def zrow(rr, c):
        def zcol(i, c2):
            wbuf[rr, pl.ds(i * 16, 16)] = zero16
            return c2
        return lax.fori_loop(0, d // 16, zcol, c)
    lax.fori_loop(0, R, zrow, 0)

    pltpu.sync_copy(idx_hbm.at[pl.ds(base, HPW)], idx_v)
    pltpu.sync_copy(sgn_hbm.at[pl.ds(base, HPW)], sgn_v)

    def group(g, c):
        touched = []
        for r in range(R):
            for half in range(2):
                iv = idx_v[g * R + r, pl.ds(16 * half, 16)]
                sv = sgn_v[g * R + r, pl.ds(16 * half, 16)]
                if remap:
                    iv = ((iv & (NBITS - 1)) << 7) | (iv >> 4)
                row = jnp.full((16,), r, jnp.int32)
                col = iv
                plsc.addupdate_scatter(wbuf, [row, col], sv)
                touched.append((row, col))
        pltpu.sync_copy(wbuf, w_hbm.at[wid * G + g])
        for row, col in touched:
            plsc.store_scatter(wbuf, [row, col], zero16)
        return c
    lax.fori_loop(0, G, group, 0)


def _make_build(d, remap):
    return pl.kernel(
        functools.partial(_build_body, d=d, remap=remap),
        out_type=jax.ShapeDtypeStruct((H // 8, 8, d), jnp.float32),
        mesh=_mesh,
        compiler_params=pltpu.CompilerParams(needs_layout_passes=False),
        scratch_types=[
            pltpu.VMEM((HPW, K), jnp.int32),
            pltpu.VMEM((HPW, K), jnp.float32),
            pltpu.VMEM((R, d), jnp.float32),
        ],
    )


_build0 = _make_build(ENC, True)
_build1 = _make_build(H, False)


def _encode_body(x_ref, out_ref):
    x = x_ref[...]
    for t in range(NBITS):
        th = (t + 0.5) / NBITS
        out_ref[:, t * F:(t + 1) * F] = (x >= th).astype(jnp.bfloat16)


def _mm0_body(enc_ref, w_ref, out_ref):
    hb = w_ref.shape[0] * 8
    w = w_ref[...].reshape(hb, ENC)
    s = lax.dot_general(enc_ref[...], w.astype(jnp.bfloat16),
                        (((1,), (1,)), ((), ())),
                        preferred_element_type=jnp.float32)
    out_ref[...] = (s >= THRESH).astype(jnp.float32)


def _mm1_out_body(a0f_ref, a0b_ref, w_ref, o0_ref, o1_ref, out_ref):
    j = pl.program_id(0)
    hb = w_ref.shape[0] * 8
    w = w_ref[...].reshape(hb, H)
    s = lax.dot_general(a0f_ref[...].astype(jnp.bfloat16),
                        w.astype(jnp.bfloat16),
                        (((1,), (1,)), ((), ())),
                        preferred_element_type=jnp.float32)
    a1b = (s >= THRESH).astype(jnp.float32)
    o0 = o0_ref[...].reshape(C, hb)
    o1 = o1_ref[...].reshape(C, hb)
    part = (lax.dot_general(a0b_ref[...], o0, (((1,), (1,)), ((), ())),
                            preferred_element_type=jnp.float32)
            + lax.dot_general(a1b, o1, (((1,), (1,)), ((), ())),
                              preferred_element_type=jnp.float32))

    @pl.when(j == 0)
    def _init():
        out_ref[...] = part

    @pl.when(j > 0)
    def _acc():
        out_ref[...] += part


def kernel(x, idx0, sgn0, idx1, sgn1, outW):
    outWt = jnp.transpose(outW, (0, 2, 1))
    w0t = _build0(idx0.astype(jnp.int32), sgn0)
    w1t = _build1(idx1.astype(jnp.int32), sgn1)
    enc = pl.pallas_call(
        _encode_body,
        out_shape=jax.ShapeDtypeStruct((B, ENC), jnp.bfloat16),
    )(x)
    hb = 512
    a0 = pl.pallas_call(
        _mm0_body,
        grid=(H // hb,),
        in_specs=[
            pl.BlockSpec((B, ENC), lambda j: (0, 0)),
            pl.BlockSpec((hb // 8, 8, ENC), lambda j: (j, 0, 0)),
        ],
        out_specs=pl.BlockSpec((B, hb), lambda j: (0, j)),
        out_shape=jax.ShapeDtypeStruct((B, H), jnp.float32),
    )(enc, w0t)
    out = pl.pallas_call(
        _mm1_out_body,
        grid=(H // hb,),
        in_specs=[
            pl.BlockSpec((B, H), lambda j: (0, 0)),
            pl.BlockSpec((B, hb), lambda j: (0, j)),
            pl.BlockSpec((hb // 8, 8, H), lambda j: (j, 0, 0)),
            pl.BlockSpec((1, C, hb), lambda j: (0, 0, j)),
            pl.BlockSpec((1, C, hb), lambda j: (1, 0, j)),
        ],
        out_specs=pl.BlockSpec((B, C), lambda j: (0, 0)),
        out_shape=jax.ShapeDtypeStruct((B, C), jnp.float32),
    )(a0, a0, w1t, outWt, outWt)
    return out
